# baseline (speedup 1.0000x reference)
import functools

import jax
import jax.numpy as jnp
from jax import lax
from jax.experimental import pallas as pl
from jax.experimental.pallas import tpu as pltpu
from jax.experimental.pallas import tpu_sc as plsc

N, E, D = 10000, 320000, 128
M = 2 * D

NC, NS = 2, 16
NW = NC * NS

NSL = 2
ESL = E // NSL


def _nf_body(n_ref, w_ref, o_ref):
    nf = jnp.dot(n_ref[...], w_ref[...], preferred_element_type=jnp.float32)
    lo = jax.lax.bitcast_convert_type(nf[:, :D].astype(jnp.bfloat16),
                                      jnp.uint16).astype(jnp.uint32)
    hi = jax.lax.bitcast_convert_type(nf[:, D:].astype(jnp.bfloat16),
                                      jnp.uint16).astype(jnp.uint32)
    o_ref[...] = jax.lax.bitcast_convert_type(lo | (hi << 16), jnp.int32)


def _tc_nf(nodes, W_nodeT):
    return pl.pallas_call(
        _nf_body,
        out_shape=jax.ShapeDtypeStruct((N, D), jnp.int32),
    )(nodes, W_nodeT)


CG = 40
EPW = ESL // NW
NCHG = EPW // CG


def _sc_gather(nf, src, dst):
    mesh = plsc.VectorSubcoreMesh(core_axis_name="c", subcore_axis_name="s")

    @functools.partial(
        pl.kernel,
        mesh=mesh,
        out_type=(jax.ShapeDtypeStruct((ESL, D), jnp.int32),
                  jax.ShapeDtypeStruct((ESL, D), jnp.int32)),
        scratch_types=(
            [pltpu.VMEM((NCHG, CG), jnp.int32)] * 2
            + [pltpu.VMEM((CG, D), jnp.int32)] * 4
            + [pltpu.SemaphoreType.DMA] * 4
        ),
    )
    def k(nf_hbm, src_hbm, dst_hbm, e1_hbm, e2_hbm,
          si_all, di_all, r1a, r2a, r1b, r2b, g0, g1, w0, w1):
        slots = ((r1a, r2a, g0, w0), (r1b, r2b, g1, w1))

        wid = lax.axis_index("s") * NC + lax.axis_index("c")
        base0 = wid * EPW
        pltpu.sync_copy(src_hbm.at[wid], si_all)
        pltpu.sync_copy(dst_hbm.at[wid], di_all)

        def g_start(c, s):
            r1, r2, g, _ = slots[s]
            pltpu.async_copy(nf_hbm.at[si_all.at[c]], r1, g)
            pltpu.async_copy(nf_hbm.at[di_all.at[c]], r2, g)

        def g_wait(s):
            r1, r2, g, _ = slots[s]
            pltpu.make_async_copy(nf_hbm.at[si_all.at[0]], r1, g).wait()
            pltpu.make_async_copy(nf_hbm.at[di_all.at[0]], r2, g).wait()

        def w_start(c, s):
            r1, r2, _, w = slots[s]
            pltpu.async_copy(r1, e1_hbm.at[pl.ds(base0 + c * CG, CG)], w)
            pltpu.async_copy(r2, e2_hbm.at[pl.ds(base0 + c * CG, CG)], w)

        def w_wait(s):
            r1, r2, _, w = slots[s]
            pltpu.make_async_copy(r1, e1_hbm.at[pl.ds(0, CG)], w).wait()
            pltpu.make_async_copy(r2, e2_hbm.at[pl.ds(0, CG)], w).wait()

        def run_chunk(c, s, static):
            g_wait(s)
            w_start(c, s)
            if static:
                if c + 2 < NCHG:
                    w_wait(s)
                    g_start(c + 2, s)
            else:
                @pl.when(c + 2 < NCHG)
                def _():
                    w_wait(s)
                    g_start(c + 2, s)

        g_start(0, 0)
        g_start(1, 1)

        def body(j, carry):
            for s in (0, 1):
                run_chunk(2 * j + s, s, False)
            return carry

        lax.fori_loop(0, NCHG // 2, body, 0)
        if NCHG % 2:
            run_chunk(NCHG - 1, (NCHG - 1) % 2, True)
        w_wait(0)
        w_wait(1)

    return k(nf, src, dst)


BE = 2000


def _unpack2(p):
    u = jax.lax.bitcast_convert_type(p, jnp.uint32)
    lo = jax.lax.bitcast_convert_type(u << 16, jnp.float32)
    hi = jax.lax.bitcast_convert_type(u & jnp.uint32(0xFFFF0000), jnp.float32)
    return lo, hi


def _msg_body(ef_ref, e1_ref, e2_ref, wf_ref, w2_ref, b2_ref, out_ref):
    ef = jnp.dot(ef_ref[...], wf_ref[...], preferred_element_type=jnp.float32)
    e1l, e1h = _unpack2(e1_ref[...])
    e2l, e2h = _unpack2(e2_ref[...])
    zl = jnp.tanh(ef[:, :D] + e1l + e2l)
    zh = jnp.tanh(ef[:, D:] + e1h + e2h)
    w2 = w2_ref[...]
    msg = (jnp.dot(zl, w2[:D], preferred_element_type=jnp.float32)
           + jnp.dot(zh, w2[D:], preferred_element_type=jnp.float32)
           + b2_ref[...])
    out_ref[0] = msg[:, :D]
    out_ref[1] = msg[:, D:]


def _tc_messages(edge_features, e1, e2, W_featT, W2T, b2r):
    return pl.pallas_call(
        _msg_body,
        grid=(ESL // BE,),
        in_specs=[
            pl.BlockSpec((BE, D), lambda i: (i, 0)),
            pl.BlockSpec((BE, D), lambda i: (i, 0)),
            pl.BlockSpec((BE, D), lambda i: (i, 0)),
            pl.BlockSpec((D, M), lambda i: (0, 0)),
            pl.BlockSpec((M, M), lambda i: (0, 0)),
            pl.BlockSpec((1, M), lambda i: (0, 0)),
        ],
        out_specs=pl.BlockSpec((2, BE, D), lambda i: (0, i, 0)),
        out_shape=jax.ShapeDtypeStruct((2, ESL, D), jnp.float32),
    )(edge_features, e1, e2, W_featT, W2T, b2r)


CS = 80
EPS = ESL // NS
NCHS = EPS // CS
RDS = 2
NROW = 624
NTAIL = N - NROW * NS


def _sc_scatter(msg2, sd, zeros):
    mesh = plsc.VectorSubcoreMesh(core_axis_name="c", subcore_axis_name="s")

    @functools.partial(
        pl.kernel,
        mesh=mesh,
        out_type=jax.ShapeDtypeStruct((2, N, D), jnp.float32),
        scratch_types=(
            [pltpu.VMEM((NCHS, CS), jnp.int32)]
            + [pltpu.VMEM((CS,), jnp.int32)] * (2 * RDS)
            + [pltpu.VMEM((CS, D), jnp.float32)] * RDS
            + [pltpu.VMEM_SHARED((N, D), jnp.float32)]
            + [pltpu.SemaphoreType.DMA] * (2 * RDS)
        ),
    )
    def k(msg_hbm, sd_hbm, z_hbm, out_hbm, *rest):
        sd_all = rest[0]
        idxs = rest[1:1 + 2 * RDS]
        msgs = rest[1 + 2 * RDS:1 + 3 * RDS]
        acc_s = rest[1 + 3 * RDS]
        sems = rest[2 + 3 * RDS:]
        slots = [(idxs[2 * s], idxs[2 * s + 1], msgs[s],
                  sems[2 * s], sems[2 * s + 1]) for s in range(RDS)]

        cid = lax.axis_index("c")
        sid = lax.axis_index("s")
        pltpu.sync_copy(sd_hbm.at[sid], sd_all)

        pltpu.sync_copy(z_hbm.at[pl.ds(sid * NROW, NROW)],
                        acc_s.at[pl.ds(sid * NROW, NROW)])

        @pl.when(sid == 0)
        def _():
            pltpu.sync_copy(z_hbm.at[pl.ds(NROW * NS, NTAIL)],
                            acc_s.at[pl.ds(NROW * NS, NTAIL)])

        plsc.subcore_barrier()

        def l_start(c, s):
            si, di, m, l, _ = slots[s]
            base = sid * EPS + c * CS
            pltpu.async_copy(msg_hbm.at[cid, pl.ds(base, CS)], m, l)

        def l_wait(s):
            si, di, m, l, _ = slots[s]
            pltpu.make_async_copy(msg_hbm.at[cid, pl.ds(0, CS)], m, l).wait()

        def a_start(c, s):
            si, di, m, _, a = slots[s]
            for kk in range(CS // 16):
                w = sd_all[c, pl.ds(kk * 16, 16)]
                si[pl.ds(kk * 16, 16)] = w & 0xFFFF
                di[pl.ds(kk * 16, 16)] = lax.shift_right_logical(w, 16)
            pltpu.async_copy(m, acc_s.at[di], a, add=True)
            pltpu.async_copy(m, acc_s.at[si], a, add=True)

        def a_wait(s):
            si, di, m, _, a = slots[s]
            pltpu.make_async_copy(m, acc_s.at[di], a).wait()
            pltpu.make_async_copy(m, acc_s.at[si], a).wait()

        def run_chunk(c, s, static):
            l_wait(s)
            a_start(c, s)
            if static:
                if c + RDS < NCHS:
                    a_wait(s)
                    l_start(c + RDS, s)
            else:
                @pl.when(c + RDS < NCHS)
                def _():
                    a_wait(s)
                    l_start(c + RDS, s)

        for s in range(min(RDS, NCHS)):
            l_start(s, s)

        def body(j, carry):
            for s in range(RDS):
                run_chunk(j * RDS + s, s, False)
            return carry

        lax.fori_loop(0, NCHS // RDS, body, 0)
        for c in range((NCHS // RDS) * RDS, NCHS):
            run_chunk(c, c % RDS, True)
        for s in range(min(RDS, NCHS)):
            a_wait(s)
        plsc.subcore_barrier()
        pltpu.sync_copy(acc_s.at[pl.ds(sid * NROW, NROW)],
                        out_hbm.at[cid, pl.ds(sid * NROW, NROW)])

        @pl.when(sid == 0)
        def _():
            pltpu.sync_copy(acc_s.at[pl.ds(NROW * NS, NTAIL)],
                            out_hbm.at[cid, pl.ds(NROW * NS, NTAIL)])

    return k(msg2, sd, zeros)


BN = 1000


def _gru_body(inpa_ref, inpb_ref, nodes_ref, wih_ref, whh_ref, bih_ref,
              bhh_ref, om_ref, run_ref, out_ref):
    nodes = nodes_ref[...]
    inp_lo = inpa_ref[0] + inpb_ref[0]
    inp_hi = inpa_ref[1] + inpb_ref[1]
    gi = (jnp.dot(inp_lo, wih_ref[...][:D],
                  preferred_element_type=jnp.float32)
          + jnp.dot(inp_hi, wih_ref[...][D:],
                    preferred_element_type=jnp.float32)
          + bih_ref[...])
    gh = jnp.dot(nodes, whh_ref[...], preferred_element_type=jnp.float32) \
        + bhh_ref[...]
    r = jax.nn.sigmoid(gi[:, :D] + gh[:, :D])
    z = jax.nn.sigmoid(gi[:, D:2 * D] + gh[:, D:2 * D])
    n = jnp.tanh(gi[:, 2 * D:] + r * gh[:, 2 * D:])
    new_nodes = (1.0 - z) * n + z * nodes
    mask = jnp.sum(om_ref[...] * run_ref[...], axis=1) > 0
    out_ref[...] = jnp.where(mask[:, None], new_nodes, nodes)


def _tc_gru(inpa, inpb, nodes, W_ihT, W_hhT, bihr, bhhr, om, run):
    B = om.shape[1]
    return pl.pallas_call(
        _gru_body,
        grid=(N // BN,),
        in_specs=[
            pl.BlockSpec((2, BN, D), lambda i: (0, i, 0)),
            pl.BlockSpec((2, BN, D), lambda i: (0, i, 0)),
            pl.BlockSpec((BN, D), lambda i: (i, 0)),
            pl.BlockSpec((M, 3 * D), lambda i: (0, 0)),
            pl.BlockSpec((D, 3 * D), lambda i: (0, 0)),
            pl.BlockSpec((1, 3 * D), lambda i: (0, 0)),
            pl.BlockSpec((1, 3 * D), lambda i: (0, 0)),
            pl.BlockSpec((BN, B), lambda i: (i, 0)),
            pl.BlockSpec((1, B), lambda i: (0, 0)),
        ],
        out_specs=pl.BlockSpec((BN, D), lambda i: (i, 0)),
        out_shape=jax.ShapeDtypeStruct((N, D), jnp.float32),
    )(inpa, inpb, nodes, W_ihT, W_hhT, bihr, bhhr, om, run)


def kernel(nodes, edge_features, W_node, W_feat, W2, b2, W_ih, W_hh,
           b_ih, b_hh, edge_source, edge_dest, owner_masks, running):
    W_nodeT = W_node.T
    W_featT = W_feat.T
    W2T = W2.T
    W_ihT = W_ih.T
    W_hhT = W_hh.T
    b2r = b2.reshape(1, M)
    bihr = b_ih.reshape(1, 3 * D)
    bhhr = b_hh.reshape(1, 3 * D)
    om = owner_masks.astype(jnp.int32).T
    run = running.astype(jnp.int32).reshape(1, -1)
    src = edge_source.astype(jnp.int32)
    dst = edge_dest.astype(jnp.int32)
    zeros = jnp.zeros((N, D), jnp.float32)

    nf = _tc_nf(nodes, W_nodeT)

    inps = []
    for sl in range(NSL):
        lo = sl * ESL
        src_s = lax.slice(src, (lo,), (lo + ESL,))
        dst_s = lax.slice(dst, (lo,), (lo + ESL,))
        ef_s = lax.slice(edge_features, (lo, 0), (lo + ESL, D))
        e1, e2 = _sc_gather(nf, src_s.reshape(NW, NCHG, CG),
                            dst_s.reshape(NW, NCHG, CG))
        msg2 = _tc_messages(ef_s, e1, e2, W_featT, W2T, b2r)
        sd = (src_s | (dst_s << 16)).reshape(NS, NCHS, CS)
        inps.append(_sc_scatter(msg2, sd, zeros))

    return _tc_gru(inps[0], inps[1], nodes, W_ihT, W_hhT, bihr, bhhr,
                   om, run)


# speedup vs baseline: 1.0106x; 1.0106x over previous
import functools

import jax
import jax.numpy as jnp
from jax import lax
from jax.experimental import pallas as pl
from jax.experimental.pallas import tpu as pltpu
from jax.experimental.pallas import tpu_sc as plsc

N, E, D = 10000, 320000, 128
M = 2 * D

NC, NS = 2, 16
NW = NC * NS

NSL = 2
ESL = E // NSL


def _nf_body(n_ref, w_ref, o_ref):
    nf = jnp.dot(n_ref[...], w_ref[...], preferred_element_type=jnp.float32)
    lo = jax.lax.bitcast_convert_type(nf[:, :D].astype(jnp.bfloat16),
                                      jnp.uint16).astype(jnp.uint32)
    hi = jax.lax.bitcast_convert_type(nf[:, D:].astype(jnp.bfloat16),
                                      jnp.uint16).astype(jnp.uint32)
    o_ref[...] = jax.lax.bitcast_convert_type(lo | (hi << 16), jnp.int32)


def _tc_nf(nodes, W_nodeT):
    return pl.pallas_call(
        _nf_body,
        out_shape=jax.ShapeDtypeStruct((N, D), jnp.int32),
    )(nodes, W_nodeT)


CG = 40
EPW = ESL // NW
NCHG = EPW // CG


def _sc_gather(nf, src, dst):
    mesh = plsc.VectorSubcoreMesh(core_axis_name="c", subcore_axis_name="s")

    @functools.partial(
        pl.kernel,
        mesh=mesh,
        out_type=(jax.ShapeDtypeStruct((ESL, D), jnp.int32),
                  jax.ShapeDtypeStruct((ESL, D), jnp.int32)),
        scratch_types=(
            [pltpu.VMEM((NCHG, CG), jnp.int32)] * 2
            + [pltpu.VMEM((CG, D), jnp.int32)] * 4
            + [pltpu.SemaphoreType.DMA] * 4
        ),
    )
    def k(nf_hbm, src_hbm, dst_hbm, e1_hbm, e2_hbm,
          si_all, di_all, r1a, r2a, r1b, r2b, g0, g1, w0, w1):
        slots = ((r1a, r2a, g0, w0), (r1b, r2b, g1, w1))

        wid = lax.axis_index("s") * NC + lax.axis_index("c")
        base0 = wid * EPW
        pltpu.sync_copy(src_hbm.at[wid], si_all)
        pltpu.sync_copy(dst_hbm.at[wid], di_all)

        def g_start(c, s):
            r1, r2, g, _ = slots[s]
            pltpu.async_copy(nf_hbm.at[si_all.at[c]], r1, g)
            pltpu.async_copy(nf_hbm.at[di_all.at[c]], r2, g)

        def g_wait(s):
            r1, r2, g, _ = slots[s]
            pltpu.make_async_copy(nf_hbm.at[si_all.at[0]], r1, g).wait()
            pltpu.make_async_copy(nf_hbm.at[di_all.at[0]], r2, g).wait()

        def w_start(c, s):
            r1, r2, _, w = slots[s]
            pltpu.async_copy(r1, e1_hbm.at[pl.ds(base0 + c * CG, CG)], w)
            pltpu.async_copy(r2, e2_hbm.at[pl.ds(base0 + c * CG, CG)], w)

        def w_wait(s):
            r1, r2, _, w = slots[s]
            pltpu.make_async_copy(r1, e1_hbm.at[pl.ds(0, CG)], w).wait()
            pltpu.make_async_copy(r2, e2_hbm.at[pl.ds(0, CG)], w).wait()

        def run_chunk(c, s, static):
            g_wait(s)
            w_start(c, s)
            if static:
                if c + 2 < NCHG:
                    w_wait(s)
                    g_start(c + 2, s)
            else:
                @pl.when(c + 2 < NCHG)
                def _():
                    w_wait(s)
                    g_start(c + 2, s)

        g_start(0, 0)
        g_start(1, 1)

        def body(j, carry):
            for s in (0, 1):
                run_chunk(2 * j + s, s, False)
            return carry

        lax.fori_loop(0, NCHG // 2, body, 0)
        if NCHG % 2:
            run_chunk(NCHG - 1, (NCHG - 1) % 2, True)
        w_wait(0)
        w_wait(1)

    return k(nf, src, dst)


BE = 2000


def _unpack2(p):
    u = jax.lax.bitcast_convert_type(p, jnp.uint32)
    lo = jax.lax.bitcast_convert_type(u << 16, jnp.float32)
    hi = jax.lax.bitcast_convert_type(u & jnp.uint32(0xFFFF0000), jnp.float32)
    return lo, hi


def _msg_body(ef_ref, e1_ref, e2_ref, wf_ref, w2_ref, b2_ref, out_ref):
    ef = jnp.dot(ef_ref[...], wf_ref[...], preferred_element_type=jnp.float32)
    e1l, e1h = _unpack2(e1_ref[...])
    e2l, e2h = _unpack2(e2_ref[...])
    zl = jnp.tanh(ef[:, :D] + e1l + e2l)
    zh = jnp.tanh(ef[:, D:] + e1h + e2h)
    w2 = w2_ref[...]
    msg = (jnp.dot(zl, w2[:D], preferred_element_type=jnp.float32)
           + jnp.dot(zh, w2[D:], preferred_element_type=jnp.float32)
           + b2_ref[...])
    out_ref[0] = msg[:, :D]
    out_ref[1] = msg[:, D:]


def _tc_messages(edge_features, e1, e2, W_featT, W2T, b2r):
    return pl.pallas_call(
        _msg_body,
        grid=(ESL // BE,),
        in_specs=[
            pl.BlockSpec((BE, D), lambda i: (i, 0)),
            pl.BlockSpec((BE, D), lambda i: (i, 0)),
            pl.BlockSpec((BE, D), lambda i: (i, 0)),
            pl.BlockSpec((D, M), lambda i: (0, 0)),
            pl.BlockSpec((M, M), lambda i: (0, 0)),
            pl.BlockSpec((1, M), lambda i: (0, 0)),
        ],
        out_specs=pl.BlockSpec((2, BE, D), lambda i: (0, i, 0)),
        out_shape=jax.ShapeDtypeStruct((2, ESL, D), jnp.float32),
    )(edge_features, e1, e2, W_featT, W2T, b2r)


CS = 80
EPS = ESL // NS
NCHS = EPS // CS
RDS = 2
NROW = 624
NTAIL = N - NROW * NS


def _sc_scatter(msg2, src, dst, zeros):
    mesh = plsc.VectorSubcoreMesh(core_axis_name="c", subcore_axis_name="s")

    @functools.partial(
        pl.kernel,
        mesh=mesh,
        out_type=jax.ShapeDtypeStruct((2, N, D), jnp.float32),
        scratch_types=(
            [pltpu.VMEM((CS,), jnp.int32)] * (2 * RDS)
            + [pltpu.VMEM((CS, D), jnp.float32)] * RDS
            + [pltpu.VMEM_SHARED((N, D), jnp.float32)]
            + [pltpu.SemaphoreType.DMA] * (2 * RDS)
        ),
    )
    def k(msg_hbm, src_hbm, dst_hbm, z_hbm, out_hbm, *rest):
        idxs = rest[:2 * RDS]
        msgs = rest[2 * RDS:3 * RDS]
        acc_s = rest[3 * RDS]
        sems = rest[3 * RDS + 1:]
        slots = [(idxs[2 * s], idxs[2 * s + 1], msgs[s],
                  sems[2 * s], sems[2 * s + 1]) for s in range(RDS)]

        cid = lax.axis_index("c")
        sid = lax.axis_index("s")

        pltpu.sync_copy(z_hbm.at[pl.ds(sid * NROW, NROW)],
                        acc_s.at[pl.ds(sid * NROW, NROW)])

        @pl.when(sid == 0)
        def _():
            pltpu.sync_copy(z_hbm.at[pl.ds(NROW * NS, NTAIL)],
                            acc_s.at[pl.ds(NROW * NS, NTAIL)])

        plsc.subcore_barrier()

        def l_start(c, s):
            si, di, m, l, _ = slots[s]
            base = sid * EPS + c * CS
            pltpu.async_copy(src_hbm.at[pl.ds(base, CS)], si, l)
            pltpu.async_copy(dst_hbm.at[pl.ds(base, CS)], di, l)
            pltpu.async_copy(msg_hbm.at[cid, pl.ds(base, CS)], m, l)

        def l_wait(s):
            si, di, m, l, _ = slots[s]
            pltpu.make_async_copy(src_hbm.at[pl.ds(0, CS)], si, l).wait()
            pltpu.make_async_copy(dst_hbm.at[pl.ds(0, CS)], di, l).wait()
            pltpu.make_async_copy(msg_hbm.at[cid, pl.ds(0, CS)], m, l).wait()

        def a_start(s):
            si, di, m, _, a = slots[s]
            pltpu.async_copy(m, acc_s.at[di], a, add=True)
            pltpu.async_copy(m, acc_s.at[si], a, add=True)

        def a_wait(s):
            si, di, m, _, a = slots[s]
            pltpu.make_async_copy(m, acc_s.at[di], a).wait()
            pltpu.make_async_copy(m, acc_s.at[si], a).wait()

        def run_chunk(c, s, static):
            l_wait(s)
            a_start(s)
            if static:
                if c + RDS < NCHS:
                    a_wait(s)
                    l_start(c + RDS, s)
            else:
                @pl.when(c + RDS < NCHS)
                def _():
                    a_wait(s)
                    l_start(c + RDS, s)

        for s in range(min(RDS, NCHS)):
            l_start(s, s)

        def body(j, carry):
            for s in range(RDS):
                run_chunk(j * RDS + s, s, False)
            return carry

        lax.fori_loop(0, NCHS // RDS, body, 0)
        for c in range((NCHS // RDS) * RDS, NCHS):
            run_chunk(c, c % RDS, True)
        for s in range(min(RDS, NCHS)):
            a_wait(s)
        plsc.subcore_barrier()
        pltpu.sync_copy(acc_s.at[pl.ds(sid * NROW, NROW)],
                        out_hbm.at[cid, pl.ds(sid * NROW, NROW)])

        @pl.when(sid == 0)
        def _():
            pltpu.sync_copy(acc_s.at[pl.ds(NROW * NS, NTAIL)],
                            out_hbm.at[cid, pl.ds(NROW * NS, NTAIL)])

    return k(msg2, src, dst, zeros)


BN = 1000


def _gru_body(inpa_ref, inpb_ref, nodes_ref, wih_ref, whh_ref, bih_ref,
              bhh_ref, om_ref, run_ref, out_ref):
    nodes = nodes_ref[...]
    inp_lo = inpa_ref[0] + inpb_ref[0]
    inp_hi = inpa_ref[1] + inpb_ref[1]
    gi = (jnp.dot(inp_lo, wih_ref[...][:D],
                  preferred_element_type=jnp.float32)
          + jnp.dot(inp_hi, wih_ref[...][D:],
                    preferred_element_type=jnp.float32)
          + bih_ref[...])
    gh = jnp.dot(nodes, whh_ref[...], preferred_element_type=jnp.float32) \
        + bhh_ref[...]
    r = jax.nn.sigmoid(gi[:, :D] + gh[:, :D])
    z = jax.nn.sigmoid(gi[:, D:2 * D] + gh[:, D:2 * D])
    n = jnp.tanh(gi[:, 2 * D:] + r * gh[:, 2 * D:])
    new_nodes = (1.0 - z) * n + z * nodes
    mask = jnp.sum(om_ref[...] * run_ref[...], axis=1) > 0
    out_ref[...] = jnp.where(mask[:, None], new_nodes, nodes)


def _tc_gru(inpa, inpb, nodes, W_ihT, W_hhT, bihr, bhhr, om, run):
    B = om.shape[1]
    return pl.pallas_call(
        _gru_body,
        grid=(N // BN,),
        in_specs=[
            pl.BlockSpec((2, BN, D), lambda i: (0, i, 0)),
            pl.BlockSpec((2, BN, D), lambda i: (0, i, 0)),
            pl.BlockSpec((BN, D), lambda i: (i, 0)),
            pl.BlockSpec((M, 3 * D), lambda i: (0, 0)),
            pl.BlockSpec((D, 3 * D), lambda i: (0, 0)),
            pl.BlockSpec((1, 3 * D), lambda i: (0, 0)),
            pl.BlockSpec((1, 3 * D), lambda i: (0, 0)),
            pl.BlockSpec((BN, B), lambda i: (i, 0)),
            pl.BlockSpec((1, B), lambda i: (0, 0)),
        ],
        out_specs=pl.BlockSpec((BN, D), lambda i: (i, 0)),
        out_shape=jax.ShapeDtypeStruct((N, D), jnp.float32),
    )(inpa, inpb, nodes, W_ihT, W_hhT, bihr, bhhr, om, run)


def kernel(nodes, edge_features, W_node, W_feat, W2, b2, W_ih, W_hh,
           b_ih, b_hh, edge_source, edge_dest, owner_masks, running):
    W_nodeT = W_node.T
    W_featT = W_feat.T
    W2T = W2.T
    W_ihT = W_ih.T
    W_hhT = W_hh.T
    b2r = b2.reshape(1, M)
    bihr = b_ih.reshape(1, 3 * D)
    bhhr = b_hh.reshape(1, 3 * D)
    om = owner_masks.astype(jnp.int32).T
    run = running.astype(jnp.int32).reshape(1, -1)
    src = edge_source.astype(jnp.int32)
    dst = edge_dest.astype(jnp.int32)
    zeros = jnp.zeros((N, D), jnp.float32)

    nf = _tc_nf(nodes, W_nodeT)

    inps = []
    for sl in range(NSL):
        lo = sl * ESL
        src_s = lax.slice(src, (lo,), (lo + ESL,))
        dst_s = lax.slice(dst, (lo,), (lo + ESL,))
        ef_s = lax.slice(edge_features, (lo, 0), (lo + ESL, D))
        e1, e2 = _sc_gather(nf, src_s.reshape(NW, NCHG, CG),
                            dst_s.reshape(NW, NCHG, CG))
        msg2 = _tc_messages(ef_s, e1, e2, W_featT, W2T, b2r)
        inps.append(_sc_scatter(msg2, src_s, dst_s, zeros))

    return _tc_gru(inps[0], inps[1], nodes, W_ihT, W_hhT, bihr, bhhr,
                   om, run)
